# baseline (speedup 1.0000x reference)
import functools

import jax
import jax.numpy as jnp
from jax import lax
from jax.experimental import pallas as pl
from jax.experimental.pallas import tpu as pltpu
from jax.experimental.pallas import tpu_sc as plsc

B = 4096
L = 50
D = 64
V = 1000000
NC = 2
NS = 16
NW = NC * NS
BPW = B // NW
PD = 2 * D
LP = 56
BB = 512


def _pad_transpose_body(t_ref, o_ref):
    o_ref[:, 0:D] = t_ref[...].T


def _make_pad_transpose(n_rows, blk):
    n_pad = -(-n_rows // 8) * 8
    return pl.pallas_call(
        _pad_transpose_body,
        grid=(-(-n_pad // blk),),
        in_specs=[pl.BlockSpec((D, blk), lambda i: (0, i))],
        out_specs=pl.BlockSpec((blk, PD), lambda i: (i, 0)),
        out_shape=jax.ShapeDtypeStruct((n_pad, PD), jnp.float32),
    )


_pad_style = _make_pad_transpose(1000, 1000)
_pad_posit = _make_pad_transpose(100, 128)

VB2 = 4096
K = 124 * VB2


def _pair_transpose_body(ta_ref, tb_ref, o_ref):
    o_ref[:, 0:D] = ta_ref[...].T
    o_ref[:, D:PD] = tb_ref[...].T


_pad_table_pair = pl.pallas_call(
    _pair_transpose_body,
    grid=(K // VB2,),
    in_specs=[
        pl.BlockSpec((D, VB2), lambda i: (0, i)),
        pl.BlockSpec((D, VB2), lambda i: (0, jnp.minimum(i + K // VB2,
                                                         -(-V // VB2) - 1))),
    ],
    out_specs=pl.BlockSpec((VB2, PD), lambda i: (i, 0)),
    out_shape=jax.ShapeDtypeStruct((K, PD), jnp.float32),
)


def _pad_tokens_body(t_ref, o_ref):
    o_ref[0:L, :] = t_ref[...]


_pad_tokens = pl.pallas_call(
    _pad_tokens_body,
    grid=(4,),
    in_specs=[pl.BlockSpec((L, B // 4), lambda i: (0, i))],
    out_specs=pl.BlockSpec((LP, B // 4), lambda i: (0, i)),
    out_shape=jax.ShapeDtypeStruct((LP, B), jnp.int32),
)


_mesh = plsc.VectorSubcoreMesh(core_axis_name="c", subcore_axis_name="s")


@functools.partial(
    pl.kernel,
    mesh=_mesh,
    out_type=(
        jax.ShapeDtypeStruct((L, B, PD), jnp.float32),
        jax.ShapeDtypeStruct((B, PD), jnp.float32),
    ),
    scratch_types=[
        pltpu.VMEM((LP, BPW), jnp.int32),
        pltpu.VMEM((B,), jnp.int32),
        pltpu.VMEM((BPW, PD), jnp.float32),
        pltpu.VMEM((BPW, PD), jnp.float32),
        pltpu.VMEM((BPW, PD), jnp.float32),
        pltpu.SemaphoreType.DMA,
        pltpu.SemaphoreType.DMA,
        pltpu.SemaphoreType.DMA,
        pltpu.SemaphoreType.DMA,
        pltpu.SemaphoreType.DMA,
    ],
    compiler_params=pltpu.CompilerParams(use_tc_tiling_on_sc=True,
                                         needs_layout_passes=False),
)
def _sc_gather(tokens_hbm, labels_hbm, tok_tab, sty_tab, scr_hbm, sty_hbm,
               idx_all, lab_all, slab_a, slab_b, styv,
               sem_ga, sem_gb, sem_wa, sem_wb, sem_sty):
    wid = lax.axis_index("s") * NC + lax.axis_index("c")
    b0 = wid * BPW

    pltpu.sync_copy(tokens_hbm.at[:, pl.ds(b0, BPW)], idx_all)
    pltpu.sync_copy(labels_hbm, lab_all)
    pltpu.async_copy(sty_tab.at[lab_all.at[pl.ds(b0, BPW)]], styv,
                     sem_sty).wait()
    cp_sty = pltpu.async_copy(styv, sty_hbm.at[pl.ds(b0, BPW)], sem_sty)

    pltpu.async_copy(tok_tab.at[idx_all.at[0]], slab_a, sem_ga).wait()

    def pair_body(p, carry):
        la = 2 * p

        @pl.when(p > 0)
        def _():
            pltpu.make_async_copy(slab_b, scr_hbm.at[la - 1, pl.ds(b0, BPW)],
                                  sem_wb).wait()

        cp_wa = pltpu.async_copy(slab_a, scr_hbm.at[la, pl.ds(b0, BPW)],
                                 sem_wa)
        pltpu.async_copy(tok_tab.at[idx_all.at[la + 1]], slab_b, sem_gb).wait()
        cp_wa.wait()
        pltpu.async_copy(slab_b, scr_hbm.at[la + 1, pl.ds(b0, BPW)], sem_wb)

        @pl.when(p < L // 2 - 1)
        def _():
            pltpu.async_copy(tok_tab.at[idx_all.at[la + 2]], slab_a,
                             sem_ga).wait()

        return carry

    lax.fori_loop(0, L // 2, pair_body, 0)
    pltpu.make_async_copy(slab_b, scr_hbm.at[L - 1, pl.ds(b0, BPW)],
                          sem_wb).wait()
    cp_sty.wait()


def _tc_out_body(scr_ref, sty_ref, pos_ref, tok_ref, o_ref):
    l = pl.program_id(1)
    par = tok_ref[pl.ds(l % 8, 1), :].T >= K
    x = jnp.where(par, scr_ref[0, :, D:PD], scr_ref[0, :, 0:D])
    x = x + sty_ref[:, 0:D]
    p = pos_ref[pl.ds(l, 1), 0:D]
    o_ref[0] = x.T + p.T


_tc_out = pl.pallas_call(
    _tc_out_body,
    grid=(B // BB, L),
    in_specs=[
        pl.BlockSpec((1, BB, PD), lambda jb, l: (l, jb, 0)),
        pl.BlockSpec((BB, PD), lambda jb, l: (jb, 0)),
        pl.BlockSpec((104, PD), lambda jb, l: (0, 0)),
        pl.BlockSpec((8, BB), lambda jb, l: (l // 8, jb)),
    ],
    out_specs=pl.BlockSpec((1, D, BB), lambda jb, l: (l, 0, jb)),
    out_shape=jax.ShapeDtypeStruct((L, D, B), jnp.float32),
)


def kernel(tokens, labels, token_table, style_table, posit_table):
    tokens = tokens.astype(jnp.int32)
    rows = jnp.where(tokens < K, tokens, tokens - K)
    scr, sty_rows = _sc_gather(
        _pad_tokens(rows.T),
        labels.astype(jnp.int32),
        _pad_table_pair(token_table.T, token_table.T),
        _pad_style(style_table.T),
    )
    out_t = _tc_out(scr, sty_rows, _pad_posit(posit_table.T), tokens.T)
    return out_t.transpose(2, 0, 1)


# speedup vs baseline: 1.1114x; 1.1114x over previous
import functools

import jax
import jax.numpy as jnp
from jax import lax
from jax.experimental import pallas as pl
from jax.experimental.pallas import tpu as pltpu
from jax.experimental.pallas import tpu_sc as plsc

B = 4096
L = 50
D = 64
V = 1000000
NC = 2
NS = 16
NW = NC * NS
BPW = B // NW
PD = 2 * D
LP = 56
BB = 512


def _pad_transpose_body(t_ref, o_ref):
    o_ref[:, 0:D] = t_ref[...].T


def _make_pad_transpose(n_rows, blk):
    n_pad = -(-n_rows // 8) * 8
    return pl.pallas_call(
        _pad_transpose_body,
        grid=(-(-n_pad // blk),),
        in_specs=[pl.BlockSpec((D, blk), lambda i: (0, i))],
        out_specs=pl.BlockSpec((blk, PD), lambda i: (i, 0)),
        out_shape=jax.ShapeDtypeStruct((n_pad, PD), jnp.float32),
    )


_pad_table = _make_pad_transpose(V, 8192)
_pad_style = _make_pad_transpose(1000, 1000)
_pad_posit = _make_pad_transpose(100, 128)


def _pad_tokens_body(t_ref, o_ref):
    o_ref[0:L, :] = t_ref[...]


_pad_tokens = pl.pallas_call(
    _pad_tokens_body,
    grid=(4,),
    in_specs=[pl.BlockSpec((L, B // 4), lambda i: (0, i))],
    out_specs=pl.BlockSpec((LP, B // 4), lambda i: (0, i)),
    out_shape=jax.ShapeDtypeStruct((LP, B), jnp.int32),
)


_mesh = plsc.VectorSubcoreMesh(core_axis_name="c", subcore_axis_name="s")


@functools.partial(
    pl.kernel,
    mesh=_mesh,
    out_type=(
        jax.ShapeDtypeStruct((L, B, PD), jnp.float32),
        jax.ShapeDtypeStruct((B, PD), jnp.float32),
    ),
    scratch_types=[
        pltpu.VMEM((LP, BPW), jnp.int32),
        pltpu.VMEM((B,), jnp.int32),
        pltpu.VMEM((BPW, PD), jnp.float32),
        pltpu.VMEM((BPW, PD), jnp.float32),
        pltpu.VMEM((BPW, PD), jnp.float32),
        pltpu.SemaphoreType.DMA,
        pltpu.SemaphoreType.DMA,
        pltpu.SemaphoreType.DMA,
        pltpu.SemaphoreType.DMA,
        pltpu.SemaphoreType.DMA,
    ],
    compiler_params=pltpu.CompilerParams(use_tc_tiling_on_sc=True,
                                         needs_layout_passes=False),
)
def _sc_gather(tokens_hbm, labels_hbm, tok_tab, sty_tab, scr_hbm, sty_hbm,
               idx_all, lab_all, slab_a, slab_b, styv,
               sem_ga, sem_gb, sem_wa, sem_wb, sem_sty):
    wid = lax.axis_index("s") * NC + lax.axis_index("c")
    b0 = wid * BPW

    pltpu.sync_copy(tokens_hbm.at[:, pl.ds(b0, BPW)], idx_all)
    pltpu.sync_copy(labels_hbm, lab_all)
    pltpu.async_copy(sty_tab.at[lab_all.at[pl.ds(b0, BPW)]], styv,
                     sem_sty).wait()
    cp_sty = pltpu.async_copy(styv, sty_hbm.at[pl.ds(b0, BPW)], sem_sty)

    pltpu.async_copy(tok_tab.at[idx_all.at[0]], slab_a, sem_ga).wait()

    def pair_body(p, carry):
        la = 2 * p

        @pl.when(p > 0)
        def _():
            pltpu.make_async_copy(slab_b, scr_hbm.at[la - 1, pl.ds(b0, BPW)],
                                  sem_wb).wait()

        cp_wa = pltpu.async_copy(slab_a, scr_hbm.at[la, pl.ds(b0, BPW)],
                                 sem_wa)
        pltpu.async_copy(tok_tab.at[idx_all.at[la + 1]], slab_b, sem_gb).wait()
        cp_wa.wait()
        pltpu.async_copy(slab_b, scr_hbm.at[la + 1, pl.ds(b0, BPW)], sem_wb)

        @pl.when(p < L // 2 - 1)
        def _():
            pltpu.async_copy(tok_tab.at[idx_all.at[la + 2]], slab_a,
                             sem_ga).wait()

        return carry

    lax.fori_loop(0, L // 2, pair_body, 0)
    pltpu.make_async_copy(slab_b, scr_hbm.at[L - 1, pl.ds(b0, BPW)],
                          sem_wb).wait()
    cp_sty.wait()


def _tc_out_body(scr_ref, sty_ref, pos_ref, o_ref):
    l = pl.program_id(1)
    x = scr_ref[0, :, 0:D] + sty_ref[:, 0:D]
    p = pos_ref[pl.ds(l, 1), 0:D]
    o_ref[0] = x.T + p.T


_tc_out = pl.pallas_call(
    _tc_out_body,
    grid=(B // BB, L),
    in_specs=[
        pl.BlockSpec((1, BB, PD), lambda jb, l: (l, jb, 0)),
        pl.BlockSpec((BB, PD), lambda jb, l: (jb, 0)),
        pl.BlockSpec((104, PD), lambda jb, l: (0, 0)),
    ],
    out_specs=pl.BlockSpec((1, D, BB), lambda jb, l: (l, 0, jb)),
    out_shape=jax.ShapeDtypeStruct((L, D, B), jnp.float32),
)


def kernel(tokens, labels, token_table, style_table, posit_table):
    tokens = tokens.astype(jnp.int32)
    scr, sty_rows = _sc_gather(
        _pad_tokens(tokens.T),
        labels.astype(jnp.int32),
        _pad_table(token_table.T),
        _pad_style(style_table.T),
    )
    out_t = _tc_out(scr, sty_rows, _pad_posit(posit_table.T))
    return out_t.transpose(2, 0, 1)


# speedup vs baseline: 1.3731x; 1.2355x over previous
import functools

import jax
import jax.numpy as jnp
from jax import lax
from jax.experimental import pallas as pl
from jax.experimental.pallas import tpu as pltpu
from jax.experimental.pallas import tpu_sc as plsc

B = 4096
L = 50
D = 64
V = 1000000
NC = 2
NS = 16
NW = NC * NS
BPW = B // NW
PD = 2 * D
LP = 56
BB = 1024


def _pad_transpose_body(t_ref, o_ref):
    o_ref[:, 0:D] = t_ref[...].T


def _make_pad_transpose(n_rows, blk):
    n_pad = -(-n_rows // 8) * 8
    return pl.pallas_call(
        _pad_transpose_body,
        grid=(-(-n_pad // blk),),
        in_specs=[pl.BlockSpec((D, blk), lambda i: (0, i))],
        out_specs=pl.BlockSpec((blk, PD), lambda i: (i, 0)),
        out_shape=jax.ShapeDtypeStruct((n_pad, PD), jnp.float32),
    )


_pad_table = _make_pad_transpose(V, 16384)
_pad_style = _make_pad_transpose(1000, 1000)
_pad_posit = _make_pad_transpose(100, 128)


def _pad_tokens_body(t_ref, o_ref):
    o_ref[0:L, :] = t_ref[...]


_pad_tokens = pl.pallas_call(
    _pad_tokens_body,
    grid=(4,),
    in_specs=[pl.BlockSpec((L, B // 4), lambda i: (0, i))],
    out_specs=pl.BlockSpec((LP, B // 4), lambda i: (0, i)),
    out_shape=jax.ShapeDtypeStruct((LP, B), jnp.int32),
)


_mesh = plsc.VectorSubcoreMesh(core_axis_name="c", subcore_axis_name="s")


@functools.partial(
    pl.kernel,
    mesh=_mesh,
    out_type=(
        jax.ShapeDtypeStruct((L, B, PD), jnp.float32),
        jax.ShapeDtypeStruct((B, PD), jnp.float32),
    ),
    scratch_types=[
        pltpu.VMEM((LP, BPW), jnp.int32),
        pltpu.VMEM((B,), jnp.int32),
        pltpu.VMEM((BPW, PD), jnp.float32),
        pltpu.VMEM((BPW, PD), jnp.float32),
        pltpu.VMEM((BPW, PD), jnp.float32),
        pltpu.SemaphoreType.DMA,
        pltpu.SemaphoreType.DMA,
        pltpu.SemaphoreType.DMA,
        pltpu.SemaphoreType.DMA,
        pltpu.SemaphoreType.DMA,
    ],
    compiler_params=pltpu.CompilerParams(use_tc_tiling_on_sc=True,
                                         needs_layout_passes=False),
)
def _sc_gather(tokens_hbm, labels_hbm, tok_tab, sty_tab, scr_hbm, sty_hbm,
               idx_all, lab_all, slab_a, slab_b, styv,
               sem_ga, sem_gb, sem_wa, sem_wb, sem_sty):
    wid = lax.axis_index("s") * NC + lax.axis_index("c")
    b0 = wid * BPW

    pltpu.sync_copy(tokens_hbm.at[:, pl.ds(b0, BPW)], idx_all)
    pltpu.sync_copy(labels_hbm, lab_all)
    pltpu.async_copy(sty_tab.at[lab_all.at[pl.ds(b0, BPW)]], styv,
                     sem_sty).wait()
    cp_sty = pltpu.async_copy(styv, sty_hbm.at[pl.ds(b0, BPW)], sem_sty)

    pltpu.async_copy(tok_tab.at[idx_all.at[0]], slab_a, sem_ga).wait()

    def pair_body(p, carry):
        la = 2 * p

        @pl.when(p > 0)
        def _():
            pltpu.make_async_copy(slab_b, scr_hbm.at[la - 1, pl.ds(b0, BPW)],
                                  sem_wb).wait()

        cp_wa = pltpu.async_copy(slab_a, scr_hbm.at[la, pl.ds(b0, BPW)],
                                 sem_wa)
        pltpu.async_copy(tok_tab.at[idx_all.at[la + 1]], slab_b, sem_gb).wait()
        cp_wa.wait()
        pltpu.async_copy(slab_b, scr_hbm.at[la + 1, pl.ds(b0, BPW)], sem_wb)

        @pl.when(p < L // 2 - 1)
        def _():
            pltpu.async_copy(tok_tab.at[idx_all.at[la + 2]], slab_a,
                             sem_ga).wait()

        return carry

    lax.fori_loop(0, L // 2, pair_body, 0)
    pltpu.make_async_copy(slab_b, scr_hbm.at[L - 1, pl.ds(b0, BPW)],
                          sem_wb).wait()
    cp_sty.wait()


def _tc_out_body(scr_ref, sty_ref, pos_ref, o_ref):
    l = pl.program_id(1)
    x = scr_ref[0, :, 0:D] + sty_ref[:, 0:D]
    p = pos_ref[pl.ds(l, 1), 0:D]
    o_ref[0] = x.T + p.T


_tc_out = pl.pallas_call(
    _tc_out_body,
    grid=(B // BB, L),
    in_specs=[
        pl.BlockSpec((1, BB, PD), lambda jb, l: (l, jb, 0)),
        pl.BlockSpec((BB, PD), lambda jb, l: (jb, 0)),
        pl.BlockSpec((104, PD), lambda jb, l: (0, 0)),
    ],
    out_specs=pl.BlockSpec((1, D, BB), lambda jb, l: (l, 0, jb)),
    out_shape=jax.ShapeDtypeStruct((L, D, B), jnp.float32),
)


def kernel(tokens, labels, token_table, style_table, posit_table):
    tokens = tokens.astype(jnp.int32)
    scr, sty_rows = _sc_gather(
        _pad_tokens(tokens.T),
        labels.astype(jnp.int32),
        _pad_table(token_table.T),
        _pad_style(style_table.T),
    )
    out_t = _tc_out(scr, sty_rows, _pad_posit(posit_table.T))
    return out_t.transpose(2, 0, 1)


# speedup vs baseline: 1.5442x; 1.1246x over previous
import functools

import jax
import jax.numpy as jnp
from jax import lax
from jax.experimental import pallas as pl
from jax.experimental.pallas import tpu as pltpu
from jax.experimental.pallas import tpu_sc as plsc

B = 4096
L = 50
D = 64
V = 1000000
NC = 2
NS = 16
NW = NC * NS
BPW = B // NW
PD = 2 * D
LP = 56
BB = 2048


def _pad_transpose_body(t_ref, o_ref):
    o_ref[:, 0:D] = t_ref[...].T


def _make_pad_transpose(n_rows, blk):
    n_pad = -(-n_rows // 8) * 8
    return pl.pallas_call(
        _pad_transpose_body,
        grid=(-(-n_pad // blk),),
        in_specs=[pl.BlockSpec((D, blk), lambda i: (0, i))],
        out_specs=pl.BlockSpec((blk, PD), lambda i: (i, 0)),
        out_shape=jax.ShapeDtypeStruct((n_pad, PD), jnp.float32),
    )


_pad_table = _make_pad_transpose(V, 32768)
_pad_style = _make_pad_transpose(1000, 1000)
_pad_posit = _make_pad_transpose(100, 128)


def _pad_tokens_body(t_ref, o_ref):
    o_ref[0:L, :] = t_ref[...]


_pad_tokens = pl.pallas_call(
    _pad_tokens_body,
    grid=(4,),
    in_specs=[pl.BlockSpec((L, B // 4), lambda i: (0, i))],
    out_specs=pl.BlockSpec((LP, B // 4), lambda i: (0, i)),
    out_shape=jax.ShapeDtypeStruct((LP, B), jnp.int32),
)


_mesh = plsc.VectorSubcoreMesh(core_axis_name="c", subcore_axis_name="s")


@functools.partial(
    pl.kernel,
    mesh=_mesh,
    out_type=(
        jax.ShapeDtypeStruct((L, B, PD), jnp.float32),
        jax.ShapeDtypeStruct((B, PD), jnp.float32),
    ),
    scratch_types=[
        pltpu.VMEM((LP, BPW), jnp.int32),
        pltpu.VMEM((B,), jnp.int32),
        pltpu.VMEM((BPW, PD), jnp.float32),
        pltpu.VMEM((BPW, PD), jnp.float32),
        pltpu.VMEM((BPW, PD), jnp.float32),
        pltpu.SemaphoreType.DMA,
        pltpu.SemaphoreType.DMA,
        pltpu.SemaphoreType.DMA,
        pltpu.SemaphoreType.DMA,
        pltpu.SemaphoreType.DMA,
    ],
    compiler_params=pltpu.CompilerParams(use_tc_tiling_on_sc=True,
                                         needs_layout_passes=False),
)
def _sc_gather(tokens_hbm, labels_hbm, tok_tab, sty_tab, scr_hbm, sty_hbm,
               idx_all, lab_all, slab_a, slab_b, styv,
               sem_ga, sem_gb, sem_wa, sem_wb, sem_sty):
    wid = lax.axis_index("s") * NC + lax.axis_index("c")
    b0 = wid * BPW

    pltpu.sync_copy(tokens_hbm.at[:, pl.ds(b0, BPW)], idx_all)
    pltpu.sync_copy(labels_hbm, lab_all)
    pltpu.async_copy(sty_tab.at[lab_all.at[pl.ds(b0, BPW)]], styv,
                     sem_sty).wait()
    cp_sty = pltpu.async_copy(styv, sty_hbm.at[pl.ds(b0, BPW)], sem_sty)

    pltpu.async_copy(tok_tab.at[idx_all.at[0]], slab_a, sem_ga).wait()

    def pair_body(p, carry):
        la = 2 * p

        @pl.when(p > 0)
        def _():
            pltpu.make_async_copy(slab_b, scr_hbm.at[la - 1, pl.ds(b0, BPW)],
                                  sem_wb).wait()

        cp_wa = pltpu.async_copy(slab_a, scr_hbm.at[la, pl.ds(b0, BPW)],
                                 sem_wa)
        pltpu.async_copy(tok_tab.at[idx_all.at[la + 1]], slab_b, sem_gb).wait()
        cp_wa.wait()
        pltpu.async_copy(slab_b, scr_hbm.at[la + 1, pl.ds(b0, BPW)], sem_wb)

        @pl.when(p < L // 2 - 1)
        def _():
            pltpu.async_copy(tok_tab.at[idx_all.at[la + 2]], slab_a,
                             sem_ga).wait()

        return carry

    lax.fori_loop(0, L // 2, pair_body, 0)
    pltpu.make_async_copy(slab_b, scr_hbm.at[L - 1, pl.ds(b0, BPW)],
                          sem_wb).wait()
    cp_sty.wait()


def _tc_out_body(scr_ref, sty_ref, pos_ref, o_ref):
    l = pl.program_id(1)
    x = scr_ref[0, :, 0:D] + sty_ref[:, 0:D]
    p = pos_ref[pl.ds(l, 1), 0:D]
    o_ref[0] = x.T + p.T


_tc_out = pl.pallas_call(
    _tc_out_body,
    grid=(B // BB, L),
    in_specs=[
        pl.BlockSpec((1, BB, PD), lambda jb, l: (l, jb, 0)),
        pl.BlockSpec((BB, PD), lambda jb, l: (jb, 0)),
        pl.BlockSpec((104, PD), lambda jb, l: (0, 0)),
    ],
    out_specs=pl.BlockSpec((1, D, BB), lambda jb, l: (l, 0, jb)),
    out_shape=jax.ShapeDtypeStruct((L, D, B), jnp.float32),
)


def kernel(tokens, labels, token_table, style_table, posit_table):
    tokens = tokens.astype(jnp.int32)
    scr, sty_rows = _sc_gather(
        _pad_tokens(tokens.T),
        labels.astype(jnp.int32),
        _pad_table(token_table.T),
        _pad_style(style_table.T),
    )
    out_t = _tc_out(scr, sty_rows, _pad_posit(posit_table.T))
    return out_t.transpose(2, 0, 1)


# speedup vs baseline: 1.6392x; 1.0616x over previous
import functools

import jax
import jax.numpy as jnp
from jax import lax
from jax.experimental import pallas as pl
from jax.experimental.pallas import tpu as pltpu
from jax.experimental.pallas import tpu_sc as plsc

B = 4096
L = 50
D = 64
V = 1000000
NC = 2
NS = 16
NW = NC * NS
BPW = B // NW
PD = 2 * D
LP = 56
BB = 4096


def _pad_transpose_body(t_ref, o_ref):
    o_ref[:, 0:D] = t_ref[...].T


def _make_pad_transpose(n_rows, blk):
    n_pad = -(-n_rows // 8) * 8
    return pl.pallas_call(
        _pad_transpose_body,
        grid=(-(-n_pad // blk),),
        in_specs=[pl.BlockSpec((D, blk), lambda i: (0, i))],
        out_specs=pl.BlockSpec((blk, PD), lambda i: (i, 0)),
        out_shape=jax.ShapeDtypeStruct((n_pad, PD), jnp.float32),
    )


_pad_table = _make_pad_transpose(V, 32768)
_pad_style = _make_pad_transpose(1000, 1000)
_pad_posit = _make_pad_transpose(100, 128)


def _pad_tokens_body(t_ref, o_ref):
    o_ref[0:L, :] = t_ref[...]


_pad_tokens = pl.pallas_call(
    _pad_tokens_body,
    grid=(4,),
    in_specs=[pl.BlockSpec((L, B // 4), lambda i: (0, i))],
    out_specs=pl.BlockSpec((LP, B // 4), lambda i: (0, i)),
    out_shape=jax.ShapeDtypeStruct((LP, B), jnp.int32),
)


_mesh = plsc.VectorSubcoreMesh(core_axis_name="c", subcore_axis_name="s")


@functools.partial(
    pl.kernel,
    mesh=_mesh,
    out_type=(
        jax.ShapeDtypeStruct((L, B, PD), jnp.float32),
        jax.ShapeDtypeStruct((B, PD), jnp.float32),
    ),
    scratch_types=[
        pltpu.VMEM((LP, BPW), jnp.int32),
        pltpu.VMEM((B,), jnp.int32),
        pltpu.VMEM((BPW, PD), jnp.float32),
        pltpu.VMEM((BPW, PD), jnp.float32),
        pltpu.VMEM((BPW, PD), jnp.float32),
        pltpu.SemaphoreType.DMA,
        pltpu.SemaphoreType.DMA,
        pltpu.SemaphoreType.DMA,
        pltpu.SemaphoreType.DMA,
        pltpu.SemaphoreType.DMA,
    ],
    compiler_params=pltpu.CompilerParams(use_tc_tiling_on_sc=True,
                                         needs_layout_passes=False),
)
def _sc_gather(tokens_hbm, labels_hbm, tok_tab, sty_tab, scr_hbm, sty_hbm,
               idx_all, lab_all, slab_a, slab_b, styv,
               sem_ga, sem_gb, sem_wa, sem_wb, sem_sty):
    wid = lax.axis_index("s") * NC + lax.axis_index("c")
    b0 = wid * BPW

    pltpu.sync_copy(tokens_hbm.at[:, pl.ds(b0, BPW)], idx_all)
    pltpu.sync_copy(labels_hbm, lab_all)
    pltpu.async_copy(sty_tab.at[lab_all.at[pl.ds(b0, BPW)]], styv,
                     sem_sty).wait()
    cp_sty = pltpu.async_copy(styv, sty_hbm.at[pl.ds(b0, BPW)], sem_sty)

    pltpu.async_copy(tok_tab.at[idx_all.at[0]], slab_a, sem_ga).wait()

    def pair_body(p, carry):
        la = 2 * p

        @pl.when(p > 0)
        def _():
            pltpu.make_async_copy(slab_b, scr_hbm.at[la - 1, pl.ds(b0, BPW)],
                                  sem_wb).wait()

        cp_wa = pltpu.async_copy(slab_a, scr_hbm.at[la, pl.ds(b0, BPW)],
                                 sem_wa)
        pltpu.async_copy(tok_tab.at[idx_all.at[la + 1]], slab_b, sem_gb).wait()
        cp_wa.wait()
        pltpu.async_copy(slab_b, scr_hbm.at[la + 1, pl.ds(b0, BPW)], sem_wb)

        @pl.when(p < L // 2 - 1)
        def _():
            pltpu.async_copy(tok_tab.at[idx_all.at[la + 2]], slab_a,
                             sem_ga).wait()

        return carry

    lax.fori_loop(0, L // 2, pair_body, 0)
    pltpu.make_async_copy(slab_b, scr_hbm.at[L - 1, pl.ds(b0, BPW)],
                          sem_wb).wait()
    cp_sty.wait()


def _tc_out_body(scr_ref, sty_ref, pos_ref, o_ref):
    l = pl.program_id(1)
    x = scr_ref[0, :, 0:D] + sty_ref[:, 0:D]
    p = pos_ref[pl.ds(l, 1), 0:D]
    o_ref[0] = x.T + p.T


_tc_out = pl.pallas_call(
    _tc_out_body,
    grid=(B // BB, L),
    in_specs=[
        pl.BlockSpec((1, BB, PD), lambda jb, l: (l, jb, 0)),
        pl.BlockSpec((BB, PD), lambda jb, l: (jb, 0)),
        pl.BlockSpec((104, PD), lambda jb, l: (0, 0)),
    ],
    out_specs=pl.BlockSpec((1, D, BB), lambda jb, l: (l, 0, jb)),
    out_shape=jax.ShapeDtypeStruct((L, D, B), jnp.float32),
)


def kernel(tokens, labels, token_table, style_table, posit_table):
    tokens = tokens.astype(jnp.int32)
    scr, sty_rows = _sc_gather(
        _pad_tokens(tokens.T),
        labels.astype(jnp.int32),
        _pad_table(token_table.T),
        _pad_style(style_table.T),
    )
    out_t = _tc_out(scr, sty_rows, _pad_posit(posit_table.T))
    return out_t.transpose(2, 0, 1)
